# baseline (speedup 1.0000x reference)
import functools

import jax
import jax.numpy as jnp
from jax import lax
from jax.experimental import pallas as pl
from jax.experimental.pallas import tpu as pltpu
from jax.experimental.pallas import tpu_sc as plsc

B = 16384
D = 32
WD = 128
RPW = WD // D
NC, NS = 2, 16
NW = NC * NS
BPW = B // NW
CH = 128
NCH = BPW // CH
NB = 2


@functools.cache
def _make_sc_gather():
    mesh = plsc.VectorSubcoreMesh(
        core_axis_name="c", subcore_axis_name="s", num_cores=NC, num_subcores=NS
    )

    @functools.partial(
        pl.kernel,
        out_type=[
            jax.ShapeDtypeStruct((B, WD), jnp.float32),
            jax.ShapeDtypeStruct((B, WD), jnp.float32),
        ],
        mesh=mesh,
        scratch_types=[
            pltpu.VMEM((BPW,), jnp.int32),
            pltpu.VMEM((BPW,), jnp.int32),
            pltpu.VMEM((NB, CH, WD), jnp.float32),
            pltpu.VMEM((NB, CH, WD), jnp.float32),
            pltpu.SemaphoreType.DMA,
            pltpu.SemaphoreType.DMA,
            pltpu.SemaphoreType.DMA,
            pltpu.SemaphoreType.DMA,
        ],
    )
    def sc_gather(uidx_hbm, iidx_hbm, ut_hbm, it_hbm, u_out, i_out,
                  uidx, iidx, ubuf, ibuf, us0, us1, is0, is1):
        wid = lax.axis_index("s") * NC + lax.axis_index("c")
        base = wid * BPW
        pltpu.sync_copy(uidx_hbm.at[wid], uidx)
        pltpu.sync_copy(iidx_hbm.at[wid], iidx)
        usem = [us0, us1]
        isem = [is0, is1]
        gu, gi = [], []
        for j in range(NB):
            gu.append(pltpu.async_copy(
                ut_hbm.at[uidx.at[pl.ds(j * CH, CH)]], ubuf.at[j], usem[j]))
            gi.append(pltpu.async_copy(
                it_hbm.at[iidx.at[pl.ds(j * CH, CH)]], ibuf.at[j], isem[j]))
        for j in range(NCH):
            s = j % NB
            gu[j].wait()
            pltpu.sync_copy(ubuf.at[s], u_out.at[pl.ds(base + j * CH, CH)])
            if j + NB < NCH:
                gu.append(pltpu.async_copy(
                    ut_hbm.at[uidx.at[pl.ds((j + NB) * CH, CH)]],
                    ubuf.at[s], usem[s]))
            gi[j].wait()
            pltpu.sync_copy(ibuf.at[s], i_out.at[pl.ds(base + j * CH, CH)])
            if j + NB < NCH:
                gi.append(pltpu.async_copy(
                    it_hbm.at[iidx.at[pl.ds((j + NB) * CH, CH)]],
                    ibuf.at[s], isem[s]))

    return sc_gather


BLK = 2048


def _mlp_body(uw_ref, iw_ref, um_ref, im_ref, w1u_ref, w1i_ref, b1_ref,
              w2_ref, b2_ref, w3_ref, b3_ref, o_ref):
    slot = lax.broadcasted_iota(jnp.int32, (1, WD), 1) // D
    mu = (slot == um_ref[...]).astype(jnp.float32)
    mi = (slot == im_ref[...]).astype(jnp.float32)
    h = jnp.dot(uw_ref[...] * mu, w1u_ref[...],
                preferred_element_type=jnp.float32)
    h = h + jnp.dot(iw_ref[...] * mi, w1i_ref[...],
                    preferred_element_type=jnp.float32)
    h = jnp.maximum(h + b1_ref[...], 0.0)
    h = jnp.dot(h, w2_ref[...], preferred_element_type=jnp.float32) + b2_ref[...]
    h = jnp.maximum(h, 0.0)
    z = jnp.dot(h, w3_ref[...], preferred_element_type=jnp.float32) + b3_ref[...]
    o_ref[...] = jax.nn.sigmoid(z)


def _mlp(u_w, i_w, u_mod, i_mod, w1u, w1i, b1, w2, b2, w3, b3):
    grid = (B // BLK,)
    full = lambda m: (0, 0)
    row = lambda m: (m, 0)
    return pl.pallas_call(
        _mlp_body,
        grid=grid,
        in_specs=[
            pl.BlockSpec((BLK, WD), row),
            pl.BlockSpec((BLK, WD), row),
            pl.BlockSpec((BLK, 1), row),
            pl.BlockSpec((BLK, 1), row),
            pl.BlockSpec(w1u.shape, full),
            pl.BlockSpec(w1i.shape, full),
            pl.BlockSpec(b1.shape, full),
            pl.BlockSpec(w2.shape, full),
            pl.BlockSpec(b2.shape, full),
            pl.BlockSpec(w3.shape, full),
            pl.BlockSpec(b3.shape, full),
        ],
        out_specs=pl.BlockSpec((BLK, 1), row),
        out_shape=jax.ShapeDtypeStruct((B, 1), jnp.float32),
        compiler_params=pltpu.CompilerParams(
            dimension_semantics=("arbitrary",),
        ),
    )(u_w, i_w, u_mod, i_mod, w1u, w1i, b1, w2, b2, w3, b3)


def kernel(user, item, user_table, item_table, W1, b1, W2, b2, W3, b3):
    user = user.astype(jnp.int32)
    item = item.astype(jnp.int32)
    uidx = (user // RPW).reshape(NW, BPW)
    iidx = (item // RPW).reshape(NW, BPW)
    ut_w = user_table.reshape(-1, WD)
    it_w = item_table.reshape(-1, WD)
    u_w, i_w = _make_sc_gather()(uidx, iidx, ut_w, it_w)
    return _mlp(
        u_w, i_w,
        (user % RPW).reshape(B, 1), (item % RPW).reshape(B, 1),
        jnp.tile(W1[:D], (RPW, 1)), jnp.tile(W1[D:], (RPW, 1)),
        b1.reshape(1, -1), W2, b2.reshape(1, -1), W3, b3.reshape(1, -1),
    )
